# Initial kernel scaffold; baseline (speedup 1.0000x reference)
import jax
import jax.numpy as jnp
from jax.experimental import pallas as pl


def kernel(is_object, position, boxes, obj_idx, obj_ids):
    raise NotImplementedError("write your pallas kernel here")


# speedup vs baseline: 14416.4094x; 14416.4094x over previous
import jax
import jax.numpy as jnp
from jax.experimental import pallas as pl
from jax.experimental.pallas import tpu as pltpu

N = 5000
M = 128
_INF = float("inf")
_IBIG = 2**31 - 1


def _body(obj_ref, xy_ref, bxy_ref, obj_idx_ref, obj_ids_ref,
          gt_ref, objout_ref, lives_ref, score_ref, dist_ref, work_ref):
    score_ref[...] = jax.nn.sigmoid(obj_ref[...])

    x = xy_ref[:, 0:1]
    y = xy_ref[:, 1:2]
    bx = bxy_ref[0:1, :]
    by = bxy_ref[1:2, :]
    dx = x - bx
    dy = y - by
    dist = dx * dx + dy * dy
    dist_ref[...] = dist

    eqm = obj_idx_ref[...] == obj_ids_ref[0:1, :]
    has_match = jnp.any(eqm, axis=1, keepdims=True)
    col_iota = jax.lax.broadcasted_iota(jnp.int32, (N, M), 1)
    match_j = jnp.min(jnp.where(eqm, col_iota, M), axis=1, keepdims=True)
    assigned_gt = jnp.any(eqm, axis=0, keepdims=True)

    gt_ref[...] = jnp.where(has_match, match_j, -1).astype(jnp.int32)
    lives_ref[...] = jnp.where(has_match, 20, 0).astype(jnp.int32)
    objout_ref[...] = obj_idx_ref[...]

    work_ref[...] = jnp.where(has_match, _INF, dist)

    free_pr = N - jnp.sum(has_match.astype(jnp.int32))
    free_gt = M - jnp.sum(assigned_gt.astype(jnp.int32))
    trips = jnp.minimum(free_pr, free_gt)

    colmask0 = jnp.where(assigned_gt, _INF, jnp.float32(0.0))
    flat_iota = (jax.lax.broadcasted_iota(jnp.int32, (N, M), 0) * M
                 + col_iota)
    row_iota = jax.lax.broadcasted_iota(jnp.int32, (N, 1), 0)
    col_iota1 = jax.lax.broadcasted_iota(jnp.int32, (1, M), 1)
    obj_ids_row = obj_ids_ref[0:1, :]

    def step(_, colmask):
        dm = work_ref[...] + colmask
        m = jnp.min(dm)
        flat = jnp.min(jnp.where(dm == m, flat_iota, _IBIG))
        i = flat // M
        j = flat - i * M
        rowsel = row_iota == i
        objv = jnp.max(jnp.where(col_iota1 == j, obj_ids_row,
                                 jnp.int32(-2**31)))
        gt_ref[...] = jnp.where(rowsel, j, gt_ref[...])
        objout_ref[...] = jnp.where(rowsel, objv, objout_ref[...])
        lives_ref[...] = jnp.where(rowsel, 20, lives_ref[...])
        work_ref[pl.ds(i, 1), :] = jnp.full((1, M), _INF, jnp.float32)
        return jnp.where(col_iota1 == j, _INF, colmask)

    jax.lax.fori_loop(0, trips, step, colmask0)


def kernel(is_object, position, boxes, obj_idx, obj_ids):
    n = obj_idx.shape[0]
    m = obj_ids.shape[0]
    obj = is_object[-1, 0, :, 0].reshape(n, 1)
    xy = position[-1, 0, :, :2]
    bxy = jnp.zeros((8, m), jnp.float32).at[0:2, :].set(boxes[:, :2].T)
    obj_idx_c = obj_idx.astype(jnp.int32).reshape(n, 1)
    obj_ids_p = jnp.zeros((8, m), jnp.int32).at[0, :].set(
        obj_ids.astype(jnp.int32))

    out = pl.pallas_call(
        _body,
        out_shape=[
            jax.ShapeDtypeStruct((n, 1), jnp.int32),
            jax.ShapeDtypeStruct((n, 1), jnp.int32),
            jax.ShapeDtypeStruct((n, 1), jnp.int32),
            jax.ShapeDtypeStruct((n, 1), jnp.float32),
            jax.ShapeDtypeStruct((n, m), jnp.float32),
        ],
        scratch_shapes=[pltpu.VMEM((n, m), jnp.float32)],
    )(obj, xy, bxy, obj_idx_c, obj_ids_p)

    gt_idx, obj_out, lives, score, dist = out
    return (gt_idx.reshape(n), obj_out.reshape(n), lives.reshape(n),
            score.reshape(n), dist)
